# Initial kernel scaffold; baseline (speedup 1.0000x reference)
import jax
import jax.numpy as jnp
from jax.experimental import pallas as pl


def kernel(x, weight, gamma, beta):
    raise NotImplementedError("write your pallas kernel here")


# speedup vs baseline: 1.1865x; 1.1865x over previous
import functools

import jax
import jax.numpy as jnp
from jax import lax
from jax.experimental import pallas as pl
from jax.experimental.pallas import tpu as pltpu


def _shift_lanes(x, delta):
    if delta == 0:
        return x
    k = delta % x.shape[-1]
    return jnp.concatenate([x[:, k:], x[:, :k]], axis=1)


def _conv_stats_kernel(x_ref, w_ref, y_ref, st_ref, *, D, H, W):
    HW = H * W
    S = D * HW
    xb = x_ref[...].astype(jnp.bfloat16)
    cin = xb.shape[0]

    lane = lax.broadcasted_iota(jnp.int32, (1, S), 1)
    wp = lane % W
    hp = (lane // W) % H

    def _mask01(cond):
        return jnp.where(cond, jnp.float32(1), jnp.float32(0)).astype(
            jnp.bfloat16)

    mh = {0: _mask01(hp >= 2), 1: None, 2: _mask01(hp < H - 2)}
    mw = {0: _mask01(wp >= 2), 1: None, 2: _mask01(wp < W - 2)}

    taps = []
    for kh in range(3):
        for kw in range(3):
            dh, dw = 2 * kh - 2, 2 * kw - 2
            xs = _shift_lanes(xb, dh * W + dw)
            if mh[kh] is not None:
                xs = xs * mh[kh]
            if mw[kw] is not None:
                xs = xs * mw[kw]
            taps.append(xs)

    x9 = jnp.concatenate(taps, axis=0)

    zeros_d = jnp.zeros((9 * cin, HW), jnp.bfloat16)
    kt0 = jnp.concatenate([zeros_d, x9[:, : S - HW]], axis=1)
    kt2 = jnp.concatenate([x9[:, HW:], zeros_d], axis=1)
    rhs = jnp.concatenate([kt0, x9, kt2], axis=0)

    acc = jnp.dot(w_ref[...], rhs, preferred_element_type=jnp.float32)
    y_ref[...] = acc.astype(jnp.bfloat16)
    s = jnp.sum(acc, axis=1, keepdims=True)
    q = jnp.sum(acc * acc, axis=1, keepdims=True)
    st_ref[...] = jnp.concatenate([s, q], axis=1)


def _bn_relu_kernel(y_ref, sc_ref, sh_ref, o_ref):
    o_ref[...] = jnp.maximum(
        y_ref[...].astype(jnp.float32) * sc_ref[...] + sh_ref[...], 0.0)


def kernel(x, weight, gamma, beta, eps=1e-5):
    N, Cin, D, H, W = x.shape
    Cout, _, KT, KH, KW = weight.shape
    T = KT * KH * KW
    S = D * H * W
    M = N * S

    xf = x.reshape(N, Cin, S)
    w_all = jnp.transpose(weight, (0, 2, 3, 4, 1)).reshape(
        Cout, T * Cin).astype(jnp.bfloat16)

    y, st = pl.pallas_call(
        functools.partial(_conv_stats_kernel, D=D, H=H, W=W),
        grid=(N,),
        in_specs=[
            pl.BlockSpec((None, Cin, S), lambda n: (n, 0, 0)),
            pl.BlockSpec((Cout, T * Cin), lambda n: (0, 0)),
        ],
        out_specs=[
            pl.BlockSpec((None, Cout, S), lambda n: (n, 0, 0)),
            pl.BlockSpec((None, Cout, 2), lambda n: (n, 0, 0)),
        ],
        out_shape=[
            jax.ShapeDtypeStruct((N, Cout, S), jnp.bfloat16),
            jax.ShapeDtypeStruct((N, Cout, 2), jnp.float32),
        ],
        compiler_params=pltpu.CompilerParams(
            dimension_semantics=("parallel",),
            vmem_limit_bytes=56 * 1024 * 1024),
    )(xf, w_all)

    csum = jnp.sum(st[:, :, 0], axis=0)
    csq = jnp.sum(st[:, :, 1], axis=0)
    mean = csum / M
    var = jnp.maximum(csq / M - mean * mean, 0.0)
    scale = gamma.astype(jnp.float32) * lax.rsqrt(var + eps)
    shift = beta.astype(jnp.float32) - mean * scale

    SB = 4 if S % 4 == 0 else 1
    out = pl.pallas_call(
        _bn_relu_kernel,
        grid=(N, SB),
        in_specs=[
            pl.BlockSpec((None, Cout, S // SB), lambda n, j: (n, 0, j)),
            pl.BlockSpec((Cout, 1), lambda n, j: (0, 0)),
            pl.BlockSpec((Cout, 1), lambda n, j: (0, 0)),
        ],
        out_specs=pl.BlockSpec((None, Cout, S // SB), lambda n, j: (n, 0, j)),
        out_shape=jax.ShapeDtypeStruct((N, Cout, S), jnp.float32),
        compiler_params=pltpu.CompilerParams(
            dimension_semantics=("parallel", "parallel")),
    )(y, scale[:, None], shift[:, None])

    return out.reshape(N, Cout, D, H, W)


# speedup vs baseline: 1.3425x; 1.1315x over previous
import functools

import jax
import jax.numpy as jnp
from jax import lax
from jax.experimental import pallas as pl
from jax.experimental.pallas import tpu as pltpu


def _shift_lanes(x, delta):
    if delta == 0:
        return x
    k = delta % x.shape[-1]
    return jnp.concatenate([x[:, k:], x[:, :k]], axis=1)


def _conv_stats_kernel(x_ref, w_ref, y_ref, st_ref, *, D, H, W):
    HW = H * W
    S = D * HW
    xb = x_ref[...]
    cin = xb.shape[0]

    lane = lax.broadcasted_iota(jnp.int32, (1, S), 1)
    wp = lane % W
    hp = (lane // W) % H

    def _mask01(cond):
        return jnp.where(cond, jnp.float32(1), jnp.float32(0)).astype(
            jnp.bfloat16)

    mh = {0: _mask01(hp >= 2), 1: None, 2: _mask01(hp < H - 2)}
    mw = {0: _mask01(wp >= 2), 1: None, 2: _mask01(wp < W - 2)}

    taps = []
    for kh in range(3):
        for kw in range(3):
            dh, dw = 2 * kh - 2, 2 * kw - 2
            xs = _shift_lanes(xb, dh * W + dw)
            if mh[kh] is not None:
                xs = xs * mh[kh]
            if mw[kw] is not None:
                xs = xs * mw[kw]
            taps.append(xs)

    x9 = jnp.concatenate(taps, axis=0)

    zeros_d = jnp.zeros((9 * cin, HW), jnp.bfloat16)
    kt0 = jnp.concatenate([zeros_d, x9[:, : S - HW]], axis=1)
    kt2 = jnp.concatenate([x9[:, HW:], zeros_d], axis=1)
    rhs = jnp.concatenate([kt0, x9, kt2], axis=0)

    acc = jnp.dot(w_ref[...], rhs, preferred_element_type=jnp.float32)
    y_ref[...] = acc.astype(jnp.bfloat16)
    s = jnp.sum(acc, axis=1, keepdims=True)
    q = jnp.sum(acc * acc, axis=1, keepdims=True)
    st_ref[...] = jnp.concatenate([s, q], axis=1)


def _bn_relu_kernel(y_ref, sc_ref, sh_ref, o_ref):
    o = jnp.maximum(
        y_ref[...].astype(jnp.float32) * sc_ref[...] + sh_ref[...], 0.0)
    o_ref[...] = jnp.transpose(o)


def kernel(x, weight, gamma, beta, eps=1e-5):
    N, Cin, D, H, W = x.shape
    Cout, _, KT, KH, KW = weight.shape
    T = KT * KH * KW
    S = D * H * W
    M = N * S

    xf = x.reshape(N, Cin, S).astype(jnp.bfloat16)
    w_all = jnp.transpose(weight, (0, 2, 3, 4, 1)).reshape(
        Cout, T * Cin).astype(jnp.bfloat16)

    y, st = pl.pallas_call(
        functools.partial(_conv_stats_kernel, D=D, H=H, W=W),
        grid=(N,),
        in_specs=[
            pl.BlockSpec((None, Cin, S), lambda n: (n, 0, 0)),
            pl.BlockSpec((Cout, T * Cin), lambda n: (0, 0)),
        ],
        out_specs=[
            pl.BlockSpec((None, Cout, S), lambda n: (n, 0, 0)),
            pl.BlockSpec((None, Cout, 2), lambda n: (n, 0, 0)),
        ],
        out_shape=[
            jax.ShapeDtypeStruct((N, Cout, S), jnp.bfloat16),
            jax.ShapeDtypeStruct((N, Cout, 2), jnp.float32),
        ],
        compiler_params=pltpu.CompilerParams(
            dimension_semantics=("parallel",),
            vmem_limit_bytes=56 * 1024 * 1024),
    )(xf, w_all)

    csum = jnp.sum(st[:, :, 0], axis=0)
    csq = jnp.sum(st[:, :, 1], axis=0)
    mean = csum / M
    var = jnp.maximum(csq / M - mean * mean, 0.0)
    scale = gamma.astype(jnp.float32) * lax.rsqrt(var + eps)
    shift = beta.astype(jnp.float32) - mean * scale

    SB = 4 if S % 4 == 0 else 1
    out = pl.pallas_call(
        _bn_relu_kernel,
        grid=(N, SB),
        in_specs=[
            pl.BlockSpec((None, Cout, S // SB), lambda n, j: (n, 0, j)),
            pl.BlockSpec((Cout, 1), lambda n, j: (0, 0)),
            pl.BlockSpec((Cout, 1), lambda n, j: (0, 0)),
        ],
        out_specs=pl.BlockSpec((None, S // SB, Cout), lambda n, j: (n, j, 0)),
        out_shape=jax.ShapeDtypeStruct((N, S, Cout), jnp.float32),
        compiler_params=pltpu.CompilerParams(
            dimension_semantics=("parallel", "parallel")),
    )(y, scale[:, None], shift[:, None])

    return jnp.transpose(out.reshape(N, D, H, W, Cout), (0, 4, 1, 2, 3))


# speedup vs baseline: 1.7930x; 1.3355x over previous
import functools

import jax
import jax.numpy as jnp
from jax import lax
from jax.experimental import pallas as pl
from jax.experimental.pallas import tpu as pltpu


def _shift_lanes(x, delta):
    if delta == 0:
        return x
    k = delta % x.shape[-1]
    return jnp.concatenate([x[:, k:], x[:, :k]], axis=1)


def _conv_stats_kernel(x_ref, w_ref, y_ref, st_ref, *, D, H, W):
    HW = H * W
    S = D * HW
    xb = jnp.transpose(x_ref[...]).astype(jnp.bfloat16)
    cin = xb.shape[0]

    lane = lax.broadcasted_iota(jnp.int32, (1, S), 1)
    wp = lane % W
    hp = (lane // W) % H

    def _mask01(cond):
        return jnp.where(cond, jnp.float32(1), jnp.float32(0)).astype(
            jnp.bfloat16)

    mh = {0: _mask01(hp >= 2), 1: None, 2: _mask01(hp < H - 2)}
    mw = {0: _mask01(wp >= 2), 1: None, 2: _mask01(wp < W - 2)}

    taps = []
    for kh in range(3):
        for kw in range(3):
            dh, dw = 2 * kh - 2, 2 * kw - 2
            xs = _shift_lanes(xb, dh * W + dw)
            if mh[kh] is not None:
                xs = xs * mh[kh]
            if mw[kw] is not None:
                xs = xs * mw[kw]
            taps.append(xs)

    x9 = jnp.concatenate(taps, axis=0)

    zeros_d = jnp.zeros((9 * cin, HW), jnp.bfloat16)
    kt0 = jnp.concatenate([zeros_d, x9[:, : S - HW]], axis=1)
    kt2 = jnp.concatenate([x9[:, HW:], zeros_d], axis=1)
    rhs = jnp.concatenate([kt0, x9, kt2], axis=0)

    acc = jnp.dot(w_ref[...], rhs, preferred_element_type=jnp.float32)
    y_ref[...] = acc.astype(jnp.bfloat16)
    s = jnp.sum(acc, axis=1, keepdims=True)
    q = jnp.sum(acc * acc, axis=1, keepdims=True)
    st_ref[...] = jnp.concatenate([s, q], axis=1)


def _bn_relu_kernel(y_ref, sc_ref, sh_ref, o_ref):
    o = jnp.maximum(
        y_ref[...].astype(jnp.float32) * sc_ref[...] + sh_ref[...], 0.0)
    o_ref[...] = jnp.transpose(o)


def kernel(x, weight, gamma, beta, eps=1e-5):
    N, Cin, D, H, W = x.shape
    Cout, _, KT, KH, KW = weight.shape
    T = KT * KH * KW
    S = D * H * W
    M = N * S

    xf = jnp.transpose(x, (0, 2, 3, 4, 1)).reshape(N, S, Cin)
    w_all = jnp.transpose(weight, (0, 2, 3, 4, 1)).reshape(
        Cout, T * Cin).astype(jnp.bfloat16)

    y, st = pl.pallas_call(
        functools.partial(_conv_stats_kernel, D=D, H=H, W=W),
        grid=(N,),
        in_specs=[
            pl.BlockSpec((None, S, Cin), lambda n: (n, 0, 0)),
            pl.BlockSpec((Cout, T * Cin), lambda n: (0, 0)),
        ],
        out_specs=[
            pl.BlockSpec((None, Cout, S), lambda n: (n, 0, 0)),
            pl.BlockSpec((None, Cout, 2), lambda n: (n, 0, 0)),
        ],
        out_shape=[
            jax.ShapeDtypeStruct((N, Cout, S), jnp.bfloat16),
            jax.ShapeDtypeStruct((N, Cout, 2), jnp.float32),
        ],
        compiler_params=pltpu.CompilerParams(
            dimension_semantics=("parallel",),
            vmem_limit_bytes=56 * 1024 * 1024),
    )(xf, w_all)

    csum = jnp.sum(st[:, :, 0], axis=0)
    csq = jnp.sum(st[:, :, 1], axis=0)
    mean = csum / M
    var = jnp.maximum(csq / M - mean * mean, 0.0)
    scale = gamma.astype(jnp.float32) * lax.rsqrt(var + eps)
    shift = beta.astype(jnp.float32) - mean * scale

    SB = 4 if S % 4 == 0 else 1
    out = pl.pallas_call(
        _bn_relu_kernel,
        grid=(N, SB),
        in_specs=[
            pl.BlockSpec((None, Cout, S // SB), lambda n, j: (n, 0, j)),
            pl.BlockSpec((Cout, 1), lambda n, j: (0, 0)),
            pl.BlockSpec((Cout, 1), lambda n, j: (0, 0)),
        ],
        out_specs=pl.BlockSpec((None, S // SB, Cout), lambda n, j: (n, j, 0)),
        out_shape=jax.ShapeDtypeStruct((N, S, Cout), jnp.float32),
        compiler_params=pltpu.CompilerParams(
            dimension_semantics=("parallel", "parallel")),
    )(y, scale[:, None], shift[:, None])

    return jnp.transpose(out.reshape(N, D, H, W, Cout), (0, 4, 1, 2, 3))


# speedup vs baseline: 1.7935x; 1.0003x over previous
import functools

import jax
import jax.numpy as jnp
from jax import lax
from jax.experimental import pallas as pl
from jax.experimental.pallas import tpu as pltpu


def _shift_lanes(x, delta):
    if delta == 0:
        return x
    k = delta % x.shape[-1]
    return jnp.concatenate([x[:, k:], x[:, :k]], axis=1)


def _fused_kernel(x_ref, w_ref, g_ref, b_ref, o_ref, y_scr, st_scr,
                  *, D, H, W, M, eps):
    HW = H * W
    S = D * HW
    p = pl.program_id(0)
    n = pl.program_id(1)
    k = pl.program_id(2)

    @pl.when(jnp.logical_and(p == 0, k == 0))
    def conv_phase():
        xb = jnp.transpose(x_ref[...]).astype(jnp.bfloat16)
        cin = xb.shape[0]

        lane = lax.broadcasted_iota(jnp.int32, (1, S), 1)
        wp = lane % W
        hp = (lane // W) % H

        def _mask01(cond):
            return jnp.where(cond, jnp.float32(1), jnp.float32(0)).astype(
                jnp.bfloat16)

        mh = {0: _mask01(hp >= 2), 1: None, 2: _mask01(hp < H - 2)}
        mw = {0: _mask01(wp >= 2), 1: None, 2: _mask01(wp < W - 2)}

        taps = []
        for kh in range(3):
            for kw in range(3):
                dh, dw = 2 * kh - 2, 2 * kw - 2
                xs = _shift_lanes(xb, dh * W + dw)
                if mh[kh] is not None:
                    xs = xs * mh[kh]
                if mw[kw] is not None:
                    xs = xs * mw[kw]
                taps.append(xs)

        x9 = jnp.concatenate(taps, axis=0)
        zeros_d = jnp.zeros((9 * cin, HW), jnp.bfloat16)
        kt0 = jnp.concatenate([zeros_d, x9[:, : S - HW]], axis=1)
        kt2 = jnp.concatenate([x9[:, HW:], zeros_d], axis=1)
        rhs = jnp.concatenate([kt0, x9, kt2], axis=0)

        acc = jnp.dot(w_ref[...], rhs, preferred_element_type=jnp.float32)
        y_scr[n] = acc.astype(jnp.bfloat16)
        s = jnp.sum(acc, axis=1, keepdims=True)
        q = jnp.sum(acc * acc, axis=1, keepdims=True)
        st = jnp.concatenate([s, q], axis=1)

        @pl.when(n == 0)
        def _():
            st_scr[...] = st

        @pl.when(n > 0)
        def _():
            st_scr[...] = st_scr[...] + st

    @pl.when(p == 1)
    def bn_phase():
        st = st_scr[...]
        mean = st[:, 0:1] * (1.0 / M)
        var = jnp.maximum(st[:, 1:2] * (1.0 / M) - mean * mean, 0.0)
        sc = g_ref[...] * lax.rsqrt(var + eps)
        sh = b_ref[...] - mean * sc
        SB = o_ref.shape[0]
        z = y_scr[n, :, pl.ds(pl.multiple_of(k * SB, 256), SB)].astype(
            jnp.float32)
        o = jnp.maximum(z * sc + sh, 0.0)
        o_ref[...] = jnp.transpose(o)


def kernel(x, weight, gamma, beta, eps=1e-5):
    N, Cin, D, H, W = x.shape
    Cout, _, KT, KH, KW = weight.shape
    T = KT * KH * KW
    S = D * H * W
    M = N * S

    xf = jnp.transpose(x, (0, 2, 3, 4, 1)).reshape(N, S, Cin)
    w_all = jnp.transpose(weight, (0, 2, 3, 4, 1)).reshape(
        Cout, T * Cin).astype(jnp.bfloat16)

    out = pl.pallas_call(
        functools.partial(_fused_kernel, D=D, H=H, W=W, M=M, eps=eps),
        grid=(2, N, 2),
        in_specs=[
            pl.BlockSpec((None, S, Cin),
                         lambda p, n, k: ((1 - p) * n + p * (N - 1), 0, 0)),
            pl.BlockSpec((Cout, T * Cin), lambda p, n, k: (0, 0)),
            pl.BlockSpec((Cout, 1), lambda p, n, k: (0, 0)),
            pl.BlockSpec((Cout, 1), lambda p, n, k: (0, 0)),
        ],
        out_specs=pl.BlockSpec((None, S // 2, Cout),
                               lambda p, n, k: (p * n, p * k, 0)),
        out_shape=jax.ShapeDtypeStruct((N, S, Cout), jnp.float32),
        scratch_shapes=[
            pltpu.VMEM((N, Cout, S), jnp.bfloat16),
            pltpu.VMEM((Cout, 2), jnp.float32),
        ],
        compiler_params=pltpu.CompilerParams(
            dimension_semantics=("arbitrary", "arbitrary", "arbitrary"),
            vmem_limit_bytes=62 * 1024 * 1024),
    )(xf, w_all, gamma.astype(jnp.float32)[:, None],
      beta.astype(jnp.float32)[:, None])

    return jnp.transpose(out.reshape(N, D, H, W, Cout), (0, 4, 1, 2, 3))


# speedup vs baseline: 2.1838x; 1.2176x over previous
import functools

import jax
import jax.numpy as jnp
from jax import lax
from jax.experimental import pallas as pl
from jax.experimental.pallas import tpu as pltpu


def _shift_lanes(x, delta):
    if delta == 0:
        return x
    k = delta % x.shape[-1]
    return jnp.concatenate([x[:, k:], x[:, :k]], axis=1)


def _fused_kernel(x_ref, w_ref, g_ref, b_ref, o_ref, y_scr, st_scr,
                  *, D, H, W, M, eps):
    HW = H * W
    S = D * HW
    p = pl.program_id(0)
    n = pl.program_id(1)

    @pl.when(p == 0)
    def conv_phase():
        xb = jnp.transpose(x_ref[...]).astype(jnp.bfloat16)
        cin = xb.shape[0]

        lane = lax.broadcasted_iota(jnp.int32, (1, S), 1)
        wp = lane % W
        hp = (lane // W) % H

        def _mask01(cond):
            return jnp.where(cond, jnp.float32(1), jnp.float32(0)).astype(
                jnp.bfloat16)

        mh = {0: _mask01(hp >= 2), 1: None, 2: _mask01(hp < H - 2)}
        mw = {0: _mask01(wp >= 2), 1: None, 2: _mask01(wp < W - 2)}

        taps = []
        for kh in range(3):
            for kw in range(3):
                dh, dw = 2 * kh - 2, 2 * kw - 2
                xs = _shift_lanes(xb, dh * W + dw)
                ms = [m for m in (mh[kh], mw[kw]) if m is not None]
                if len(ms) == 2:
                    xs = xs * (ms[0] * ms[1])
                elif ms:
                    xs = xs * ms[0]
                taps.append(xs)

        x9 = jnp.concatenate(taps, axis=0)
        zeros_d = jnp.zeros((9 * cin, HW), jnp.bfloat16)
        kt0 = jnp.concatenate([zeros_d, x9[:, : S - HW]], axis=1)
        kt2 = jnp.concatenate([x9[:, HW:], zeros_d], axis=1)
        rhs = jnp.concatenate([kt0, x9, kt2], axis=0)

        acc = jnp.dot(w_ref[...], rhs, preferred_element_type=jnp.float32)
        y_scr[n] = acc.astype(jnp.bfloat16)
        s = jnp.sum(acc, axis=1, keepdims=True)
        q = jnp.sum(acc * acc, axis=1, keepdims=True)
        st = jnp.concatenate([s, q], axis=1)

        @pl.when(n == 0)
        def _():
            st_scr[...] = st

        @pl.when(n > 0)
        def _():
            st_scr[...] = st_scr[...] + st

    @pl.when(p == 1)
    def bn_phase():
        st = st_scr[...]
        mean = st[:, 0:1] * (1.0 / M)
        var = jnp.maximum(st[:, 1:2] * (1.0 / M) - mean * mean, 0.0)
        sc = g_ref[...] * lax.rsqrt(var + eps)
        sh = b_ref[...] - mean * sc
        z = y_scr[n].astype(jnp.float32)
        o = jnp.maximum(z * sc + sh, 0.0)
        o_ref[...] = jnp.transpose(o)


def kernel(x, weight, gamma, beta, eps=1e-5):
    N, Cin, D, H, W = x.shape
    Cout, _, KT, KH, KW = weight.shape
    T = KT * KH * KW
    S = D * H * W
    M = N * S

    xf = jnp.transpose(x, (0, 2, 3, 4, 1)).reshape(N, S, Cin)
    w_all = jnp.transpose(weight, (0, 2, 3, 4, 1)).reshape(
        Cout, T * Cin).astype(jnp.bfloat16)

    out = pl.pallas_call(
        functools.partial(_fused_kernel, D=D, H=H, W=W, M=M, eps=eps),
        grid=(2, N),
        in_specs=[
            pl.BlockSpec((None, S, Cin),
                         lambda p, n: ((1 - p) * n + p * (N - 1), 0, 0)),
            pl.BlockSpec((Cout, T * Cin), lambda p, n: (0, 0)),
            pl.BlockSpec((Cout, 1), lambda p, n: (0, 0)),
            pl.BlockSpec((Cout, 1), lambda p, n: (0, 0)),
        ],
        out_specs=pl.BlockSpec((None, S, Cout), lambda p, n: (p * n, 0, 0)),
        out_shape=jax.ShapeDtypeStruct((N, S, Cout), jnp.float32),
        scratch_shapes=[
            pltpu.VMEM((N, Cout, S), jnp.bfloat16),
            pltpu.VMEM((Cout, 2), jnp.float32),
        ],
        compiler_params=pltpu.CompilerParams(
            dimension_semantics=("arbitrary", "arbitrary"),
            vmem_limit_bytes=63 * 1024 * 1024),
    )(xf, w_all, gamma.astype(jnp.float32)[:, None],
      beta.astype(jnp.float32)[:, None])

    return jnp.transpose(out.reshape(N, D, H, W, Cout), (0, 4, 1, 2, 3))
